# Initial kernel scaffold; baseline (speedup 1.0000x reference)
import jax
import jax.numpy as jnp
from jax.experimental import pallas as pl


def kernel(x, w1_t, b1, w3_t, b3, gamma, beta):
    raise NotImplementedError("write your pallas kernel here")


# speedup vs baseline: 3.9007x; 3.9007x over previous
import functools

import jax
import jax.numpy as jnp
from jax import lax
from jax.experimental import pallas as pl
from jax.experimental.pallas import tpu as pltpu


def _shift_lanes(a, s):
    if s > 0:
        return jnp.pad(a[:, s:], ((0, 0), (0, s)))
    if s < 0:
        return jnp.pad(a[:, :s], ((0, 0), (-s, 0)))
    return a


def _fused_conv_stats_kernel(x_ref, w1_ref, b1_ref, w3_ref, b3_ref,
                             x3_ref, st_ref, *, mid, height, width):
    hw = height * width
    xf = x_ref[0]
    x0 = jnp.dot(w1_ref[...], xf,
                 preferred_element_type=jnp.float32) + b1_ref[...]
    x1 = x0[:mid]

    pos = lax.broadcasted_iota(jnp.int32, (1, hw), 1)
    hidx = pos // width
    widx = pos - hidx * width
    taps = []
    for dh in (-1, 0, 1):
        for dw in (-1, 0, 1):
            shifted = _shift_lanes(x1, dh * width + dw)
            ok_h = jnp.logical_and(hidx + dh >= 0, hidx + dh < height)
            ok_w = jnp.logical_and(widx + dw >= 0, widx + dw < width)
            mask = jnp.logical_and(ok_h, ok_w)
            taps.append(jnp.where(mask, shifted, 0.0))
    patch = jnp.concatenate(taps, axis=0)
    y = jnp.dot(w3_ref[...], patch,
                preferred_element_type=jnp.float32) + b3_ref[...]

    x3 = jnp.concatenate([y, x0[mid:]], axis=0)
    x3_ref[0] = x3
    st_ref[0] = jnp.concatenate(
        [jnp.sum(x3, axis=1, keepdims=True),
         jnp.sum(x3 * x3, axis=1, keepdims=True)], axis=1)


def _bn_residual_kernel(x_ref, x3_ref, scale_ref, shift_ref, o_ref):
    o_ref[...] = x_ref[...] + x3_ref[...] * scale_ref[...] + shift_ref[...]


def kernel(x, w1_t, b1, w3_t, b3, gamma, beta, eps=1e-5):
    N, C, H, W = x.shape
    mid = C // 2
    HW = H * W
    M = N * HW

    xf = x.reshape(N, C, HW).astype(jnp.float32)
    w1 = w1_t[:, :, 0, 0].astype(jnp.float32)
    b1c = b1.reshape(C, 1).astype(jnp.float32)
    w3 = jnp.transpose(w3_t, (0, 2, 3, 1)).reshape(mid, 9 * mid)
    w3 = w3.astype(jnp.float32)
    b3c = b3.reshape(mid, 1).astype(jnp.float32)

    kern = functools.partial(_fused_conv_stats_kernel,
                             mid=mid, height=H, width=W)
    x3, stats = pl.pallas_call(
        kern,
        out_shape=(jax.ShapeDtypeStruct((N, C, HW), jnp.float32),
                   jax.ShapeDtypeStruct((N, C, 2), jnp.float32)),
        grid=(N,),
        in_specs=[
            pl.BlockSpec((1, C, HW), lambda n: (n, 0, 0)),
            pl.BlockSpec((C, C), lambda n: (0, 0)),
            pl.BlockSpec((C, 1), lambda n: (0, 0)),
            pl.BlockSpec((mid, 9 * mid), lambda n: (0, 0)),
            pl.BlockSpec((mid, 1), lambda n: (0, 0)),
        ],
        out_specs=(
            pl.BlockSpec((1, C, HW), lambda n: (n, 0, 0)),
            pl.BlockSpec((1, C, 2), lambda n: (n, 0, 0)),
        ),
        compiler_params=pltpu.CompilerParams(
            dimension_semantics=("parallel",)),
    )(xf, w1, b1c, w3, b3c)

    tot = jnp.sum(stats, axis=0)
    mean = tot[:, 0] / M
    var = tot[:, 1] / M - mean * mean
    scale = (gamma.astype(jnp.float32) * lax.rsqrt(var + eps)).reshape(C, 1)
    shift = (beta.astype(jnp.float32) - mean * scale[:, 0]).reshape(C, 1)

    bn = 4
    while N % bn:
        bn -= 1
    out = pl.pallas_call(
        _bn_residual_kernel,
        out_shape=jax.ShapeDtypeStruct((N, C, HW), jnp.float32),
        grid=(N // bn,),
        in_specs=[
            pl.BlockSpec((bn, C, HW), lambda i: (i, 0, 0)),
            pl.BlockSpec((bn, C, HW), lambda i: (i, 0, 0)),
            pl.BlockSpec((C, 1), lambda i: (0, 0)),
            pl.BlockSpec((C, 1), lambda i: (0, 0)),
        ],
        out_specs=pl.BlockSpec((bn, C, HW), lambda i: (i, 0, 0)),
        compiler_params=pltpu.CompilerParams(
            dimension_semantics=("parallel",)),
    )(xf, x3, scale, shift)
    return out.reshape(N, C, H, W)
